# Initial kernel scaffold; baseline (speedup 1.0000x reference)
import jax
import jax.numpy as jnp
from jax.experimental import pallas as pl


def kernel(x, edge_index, edge_types, emb, W_rel, b_rel, W_root):
    raise NotImplementedError("write your pallas kernel here")


# speedup vs baseline: 7.2948x; 7.2948x over previous
import functools

import jax
import jax.numpy as jnp
from jax import lax
from jax.experimental import pallas as pl
from jax.experimental.pallas import tpu as pltpu
from jax.experimental.pallas import tpu_sc as plsc

N = 10000
E = 320000
D = 128
T = 16

NC = 2
NS = 16
NW = NC * NS

EW = E // NS
CHUNK = 80
NCH = EW // CHUNK
HALF = NCH // 2

DH = D // NC
NPAD = 10240
ROWS_PER_TILE = NPAD // NS
ZROWS = 128
NHC = N // NC
NTC = NHC * T
NTPAD = 83968
CNT_PER_TILE = NTPAD // NS


def _sc_body(xs_hbm, src_hbm, dst_hbm, et_hbm,
             aggp_hbm, cntp_hbm,
             src_v, dst_v, key_v,
             rows_v, ones_v, zrow_v, zcnt_v,
             agg_sh, cnt_sh,
             sem):
    c = lax.axis_index("c")
    s = lax.axis_index("s")

    pltpu.sync_copy(src_hbm.at[s], src_v)
    pltpu.sync_copy(dst_hbm.at[s], dst_v)
    pltpu.sync_copy(et_hbm.at[s], key_v)

    for g in range(CHUNK // 16):
        ones_v[pl.ds(g * 16, 16)] = jnp.ones((16,), jnp.float32)

    zv = jnp.zeros((16,), jnp.float32)

    def zrow_body(i, _):
        for g in range(DH // 16):
            zrow_v[i, pl.ds(g * 16, 16)] = zv
        return 0
    lax.fori_loop(0, ZROWS, zrow_body, 0)

    def zcnt_body(i, _):
        zcnt_v[pl.ds(i * 16, 16)] = zv
        return 0
    lax.fori_loop(0, CNT_PER_TILE // 16, zcnt_body, 0)

    kbase = c * NTC

    def key_body(j, _):
        for g in range(CHUNK // 16):
            dv = dst_v[j, pl.ds(g * 16, 16)]
            ev = key_v[j, pl.ds(g * 16, 16)]
            key = dv * T + ev
            k0 = key - kbase
            valid = (k0 >= 0) & (k0 < NTC)
            sink = NTC + (key & 2047)
            key_v[j, pl.ds(g * 16, 16)] = jnp.where(valid, k0, sink)
        return 0
    lax.fori_loop(0, NCH, key_body, 0)

    for k in range(ROWS_PER_TILE // ZROWS):
        pltpu.sync_copy(zrow_v, agg_sh.at[pl.ds(s * ROWS_PER_TILE + k * ZROWS, ZROWS)])
    pltpu.sync_copy(zcnt_v, cnt_sh.at[pl.ds(s * CNT_PER_TILE, CNT_PER_TILE)])
    plsc.subcore_barrier()

    x_hbm = xs_hbm.at[c]

    def chunk_body(j, _):
        pltpu.async_copy(x_hbm.at[src_v.at[j]], rows_v, sem).wait()
        pltpu.sync_copy(rows_v, agg_sh.at[dst_v.at[j]], add=True)
        pltpu.sync_copy(ones_v, cnt_sh.at[key_v.at[j]], add=True)
        return 0
    lax.fori_loop(0, NCH, chunk_body, 0)

    plsc.subcore_barrier()

    for k in range(ROWS_PER_TILE // ZROWS):
        off = s * ROWS_PER_TILE + k * ZROWS
        pltpu.sync_copy(agg_sh.at[pl.ds(off, ZROWS)], aggp_hbm.at[c, pl.ds(off, ZROWS)])
    pltpu.sync_copy(cnt_sh.at[pl.ds(s * CNT_PER_TILE, CNT_PER_TILE)],
                    cntp_hbm.at[pl.ds(c * NTPAD + s * CNT_PER_TILE, CNT_PER_TILE)])


_sc_scatter = functools.partial(
    pl.kernel,
    out_type=(
        jax.ShapeDtypeStruct((NC, NPAD, DH), jnp.float32),
        jax.ShapeDtypeStruct((NC * NTPAD,), jnp.float32),
    ),
    mesh=plsc.VectorSubcoreMesh(core_axis_name="c", subcore_axis_name="s"),
    compiler_params=pltpu.CompilerParams(use_tc_tiling_on_sc=False),
    scratch_types=[
        pltpu.VMEM((NCH, CHUNK), jnp.int32),
        pltpu.VMEM((NCH, CHUNK), jnp.int32),
        pltpu.VMEM((NCH, CHUNK), jnp.int32),
        pltpu.VMEM((CHUNK, DH), jnp.float32),
        pltpu.VMEM((CHUNK,), jnp.float32),
        pltpu.VMEM((ZROWS, DH), jnp.float32),
        pltpu.VMEM((CNT_PER_TILE,), jnp.float32),
        pltpu.VMEM_SHARED((NPAD, DH), jnp.float32),
        pltpu.VMEM_SHARED((NTPAD,), jnp.float32),
        pltpu.SemaphoreType.DMA,
    ],
)(_sc_body)


BLK = 2000


def _tc_body(aggp_ref, cntp_ref, x_ref, emb_ref, wrel_ref, brel_ref, wroot_ref,
             out_ref):
    agg = jnp.concatenate([aggp_ref[0], aggp_ref[1]], axis=1)
    cnt = cntp_ref[...]
    embw = jnp.dot(emb_ref[...], wrel_ref[...],
                   preferred_element_type=jnp.float32)
    acc = jnp.dot(agg, wrel_ref[...], preferred_element_type=jnp.float32)
    acc += jnp.dot(cnt, embw, preferred_element_type=jnp.float32)
    acc += jnp.dot(x_ref[...], wroot_ref[...], preferred_element_type=jnp.float32)
    out_ref[...] = acc + brel_ref[...]


def _tc_dense(aggp, cntp, x, emb, W_rel, b_rel, W_root):
    grid = (N // BLK,)
    return pl.pallas_call(
        _tc_body,
        grid=grid,
        in_specs=[
            pl.BlockSpec((NC, BLK, DH), lambda i: (0, i, 0)),
            pl.BlockSpec((BLK, T), lambda i: (i, 0)),
            pl.BlockSpec((BLK, D), lambda i: (i, 0)),
            pl.BlockSpec((T, D), lambda i: (0, 0)),
            pl.BlockSpec((D, D), lambda i: (0, 0)),
            pl.BlockSpec((1, D), lambda i: (0, 0)),
            pl.BlockSpec((D, D), lambda i: (0, 0)),
        ],
        out_specs=pl.BlockSpec((BLK, D), lambda i: (i, 0)),
        out_shape=jax.ShapeDtypeStruct((N, D), jnp.float32),
    )(aggp, cntp, x, emb, W_rel, b_rel, W_root)


def kernel(x, edge_index, edge_types, emb, W_rel, b_rel, W_root):
    xs = jnp.stack([x[:, :DH], x[:, DH:]])
    src = edge_index[0].reshape(NS, NCH, CHUNK)
    dst = edge_index[1].reshape(NS, NCH, CHUNK)
    et = edge_types.reshape(NS, NCH, CHUNK)
    aggp, cntp = _sc_scatter(xs, src, dst, et)
    cntp = cntp.reshape(NC, NTPAD)[:, :NTC].reshape(N, T)
    return _tc_dense(aggp, cntp, x, emb, W_rel, b_rel.reshape(1, D), W_root)
